# Initial kernel scaffold; baseline (speedup 1.0000x reference)
import functools

import jax
import jax.numpy as jnp
from jax import lax
from jax.experimental import pallas as pl
from jax.experimental.pallas import tpu as pltpu
from jax.experimental.pallas import tpu_sc as plsc

N_NODES = 10000
N_EDGES = 320000
D = 128
ED = 16

_NODE_BLK = 2000
_EDGE_BLK = 2000
_CH = 80


def _proj_body(h_ref, w1a_ref, w1b_ref, b1_ref, a_ref, b_ref):
    h = h_ref[...]
    a_ref[...] = lax.dot_general(h, w1a_ref[...], (((1,), (1,)), ((), ())),
                                 preferred_element_type=jnp.float32) + b1_ref[...]
    b_ref[...] = lax.dot_general(h, w1b_ref[...], (((1,), (1,)), ((), ())),
                                 preferred_element_type=jnp.float32)


def _msg_body(ssrc_ref, sdst_ref, e_ref, w1c_ref, w2_ref, b2_ref, out_ref):
    pre = ssrc_ref[...] + sdst_ref[...] + lax.dot_general(
        e_ref[...], w1c_ref[...], (((1,), (1,)), ((), ())),
        preferred_element_type=jnp.float32)
    hidden = jnp.maximum(pre, 0.0)
    out_ref[...] = lax.dot_general(hidden, w2_ref[...], (((1,), (1,)), ((), ())),
                                   preferred_element_type=jnp.float32) + b2_ref[...]


def _update_body(h_ref, p_ref, w3a_ref, w3b_ref, b3_ref, out_ref):
    agg = p_ref[0] + p_ref[1]
    pre = (lax.dot_general(h_ref[...], w3a_ref[...], (((1,), (1,)), ((), ())),
                           preferred_element_type=jnp.float32)
           + lax.dot_general(agg, w3b_ref[...], (((1,), (1,)), ((), ())),
                             preferred_element_type=jnp.float32)
           + b3_ref[...])
    out_ref[...] = jnp.maximum(pre, 0.0)


def _proj(h, w1a, w1b, b1r):
    nb = N_NODES // _NODE_BLK
    full = lambda i: (0, 0)
    return pl.pallas_call(
        _proj_body,
        grid=(nb,),
        in_specs=[pl.BlockSpec((_NODE_BLK, D), lambda i: (i, 0)),
                  pl.BlockSpec((D, D), full),
                  pl.BlockSpec((D, D), full),
                  pl.BlockSpec((1, D), full)],
        out_specs=[pl.BlockSpec((_NODE_BLK, D), lambda i: (i, 0)),
                   pl.BlockSpec((_NODE_BLK, D), lambda i: (i, 0))],
        out_shape=[jax.ShapeDtypeStruct((N_NODES, D), jnp.float32),
                   jax.ShapeDtypeStruct((N_NODES, D), jnp.float32)],
    )(h, w1a, w1b, b1r)


def _msg(ssrc, sdst, e, w1c, w2, b2r):
    nb = N_EDGES // _EDGE_BLK
    full = lambda i: (0, 0)
    return pl.pallas_call(
        _msg_body,
        grid=(nb,),
        in_specs=[pl.BlockSpec((_EDGE_BLK, D), lambda i: (i, 0)),
                  pl.BlockSpec((_EDGE_BLK, D), lambda i: (i, 0)),
                  pl.BlockSpec((_EDGE_BLK, ED), lambda i: (i, 0)),
                  pl.BlockSpec((D, ED), full),
                  pl.BlockSpec((D, D), full),
                  pl.BlockSpec((1, D), full)],
        out_specs=pl.BlockSpec((_EDGE_BLK, D), lambda i: (i, 0)),
        out_shape=jax.ShapeDtypeStruct((N_EDGES, D), jnp.float32),
    )(ssrc, sdst, e, w1c, w2, b2r)


def _update(h, parts, w3a, w3b, b3r):
    nb = N_NODES // _NODE_BLK
    full = lambda i: (0, 0)
    nparts = parts.shape[0]
    return pl.pallas_call(
        _update_body,
        grid=(nb,),
        in_specs=[pl.BlockSpec((_NODE_BLK, D), lambda i: (i, 0)),
                  pl.BlockSpec((nparts, _NODE_BLK, D), lambda i: (0, i, 0)),
                  pl.BlockSpec((D, D), full),
                  pl.BlockSpec((D, D), full),
                  pl.BlockSpec((1, D), full)],
        out_specs=pl.BlockSpec((_NODE_BLK, D), lambda i: (i, 0)),
        out_shape=jax.ShapeDtypeStruct((N_NODES, D), jnp.float32),
    )(h, parts, w3a, w3b, b3r)


def _sc_info():
    try:
        info = plsc.get_sparse_core_info()
        return info.num_cores, info.num_subcores
    except Exception:
        return 2, 16


@functools.cache
def _make_gather():
    nc, ns = _sc_info()
    nw = nc * ns
    epw = N_EDGES // nw
    nch = epw // _CH
    mesh = plsc.VectorSubcoreMesh(core_axis_name="c", subcore_axis_name="s")

    @functools.partial(
        pl.kernel, mesh=mesh,
        out_type=(jax.ShapeDtypeStruct((N_EDGES, D), jnp.float32),
                  jax.ShapeDtypeStruct((N_EDGES, D), jnp.float32)),
        scratch_types=[pltpu.VMEM((_CH,), jnp.int32),
                       pltpu.VMEM((_CH,), jnp.int32),
                       pltpu.VMEM((_CH, D), jnp.float32),
                       pltpu.VMEM((_CH, D), jnp.float32),
                       pltpu.SemaphoreType.DMA,
                       pltpu.SemaphoreType.DMA])
    def gather_k(a_hbm, b_hbm, src_hbm, dst_hbm, osrc_hbm, odst_hbm,
                 sidx, didx, abuf, bbuf, sema, semb):
        wid = lax.axis_index("s") * nc + lax.axis_index("c")
        base = wid * epw

        def body(i, carry):
            off = base + i * _CH
            pltpu.sync_copy(src_hbm.at[pl.ds(off, _CH)], sidx)
            pltpu.sync_copy(dst_hbm.at[pl.ds(off, _CH)], didx)
            cpa = pltpu.async_copy(a_hbm.at[sidx], abuf, sema)
            cpb = pltpu.async_copy(b_hbm.at[didx], bbuf, semb)
            cpa.wait()
            cpb.wait()
            pltpu.sync_copy(abuf, osrc_hbm.at[pl.ds(off, _CH)])
            pltpu.sync_copy(bbuf, odst_hbm.at[pl.ds(off, _CH)])
            return carry

        lax.fori_loop(0, nch, body, 0)

    return gather_k


@functools.cache
def _make_scatter():
    nc, ns = _sc_info()
    nw = nc * ns
    epw = N_EDGES // nw
    nch = epw // _CH
    rpt = N_NODES // ns
    mesh = plsc.VectorSubcoreMesh(core_axis_name="c", subcore_axis_name="s")

    @functools.partial(
        pl.kernel, mesh=mesh,
        out_type=jax.ShapeDtypeStruct((nc, N_NODES, D), jnp.float32),
        scratch_types=[pltpu.VMEM((_CH,), jnp.int32),
                       pltpu.VMEM((_CH, D), jnp.float32),
                       pltpu.VMEM((rpt, D), jnp.float32),
                       pltpu.VMEM_SHARED((N_NODES, D), jnp.float32)])
    def scatter_k(msg_hbm, dst_hbm, out_hbm, didx, mbuf, tbuf, acc):
        c = lax.axis_index("c")
        s = lax.axis_index("s")
        lanes = lax.iota(jnp.int32, 16)
        zero16 = jnp.zeros((16,), jnp.float32)

        def zbody(i, carry):
            flat = i * 16 + lanes
            rows = lax.shift_right_logical(flat, 7)
            cols = jnp.bitwise_and(flat, 127)
            plsc.store_scatter(tbuf, [rows, cols], zero16)
            return carry

        lax.fori_loop(0, rpt * D // 16, zbody, 0)
        pltpu.sync_copy(tbuf, acc.at[pl.ds(s * rpt, rpt)])
        plsc.subcore_barrier()

        wid = s * nc + c
        base = wid * epw

        def body(i, carry):
            off = base + i * _CH
            pltpu.sync_copy(dst_hbm.at[pl.ds(off, _CH)], didx)
            pltpu.sync_copy(msg_hbm.at[pl.ds(off, _CH)], mbuf)
            pltpu.sync_copy(mbuf, acc.at[didx], add=True)
            return carry

        lax.fori_loop(0, nch, body, 0)
        plsc.subcore_barrier()

        pltpu.sync_copy(acc.at[pl.ds(s * rpt, rpt)], tbuf)
        pltpu.sync_copy(tbuf, out_hbm.at[c, pl.ds(s * rpt, rpt)])

    return scatter_k


def kernel(h, edge_index, e, W1, b1, W2, b2, W3, b3):
    src = edge_index[0]
    dst = edge_index[1]
    w1a = W1[:, :D]
    w1b = W1[:, D:2 * D]
    w1c = W1[:, 2 * D:]
    w3a = W3[:, :D]
    w3b = W3[:, D:]
    b1r = b1.reshape(1, D)
    b2r = b2.reshape(1, D)
    b3r = b3.reshape(1, D)

    a, b = _proj(h, w1a, w1b, b1r)
    ssrc, sdst = _make_gather()(a, b, src, dst)
    msgs = _msg(ssrc, sdst, e, w1c, W2, b2r)
    parts = _make_scatter()(msgs, dst)
    return _update(h, parts, w3a, w3b, b3r)


# speedup vs baseline: 2.6940x; 2.6940x over previous
import functools

import jax
import jax.numpy as jnp
from jax import lax
from jax.experimental import pallas as pl
from jax.experimental.pallas import tpu as pltpu
from jax.experimental.pallas import tpu_sc as plsc

N_NODES = 10000
N_EDGES = 320000
D = 128
ED = 16

_NODE_BLK = 2000
_EDGE_BLK = 2000
_CH = 80


def _proj_body(h_ref, w1a_ref, w1b_ref, b1_ref, a_ref, b_ref):
    h = h_ref[...]
    a_ref[...] = lax.dot_general(h, w1a_ref[...], (((1,), (1,)), ((), ())),
                                 preferred_element_type=jnp.float32) + b1_ref[...]
    b_ref[...] = lax.dot_general(h, w1b_ref[...], (((1,), (1,)), ((), ())),
                                 preferred_element_type=jnp.float32)


def _msg_body(ssrc_ref, sdst_ref, e_ref, w1c_ref, w2_ref, b2_ref, out_ref):
    pre = ssrc_ref[...] + sdst_ref[...] + lax.dot_general(
        e_ref[...], w1c_ref[...], (((1,), (1,)), ((), ())),
        preferred_element_type=jnp.float32)
    hidden = jnp.maximum(pre, 0.0)
    out_ref[...] = lax.dot_general(hidden, w2_ref[...], (((1,), (1,)), ((), ())),
                                   preferred_element_type=jnp.float32) + b2_ref[...]


def _update_body(h_ref, p_ref, w3a_ref, w3b_ref, b3_ref, out_ref):
    agg = p_ref[0] + p_ref[1]
    pre = (lax.dot_general(h_ref[...], w3a_ref[...], (((1,), (1,)), ((), ())),
                           preferred_element_type=jnp.float32)
           + lax.dot_general(agg, w3b_ref[...], (((1,), (1,)), ((), ())),
                             preferred_element_type=jnp.float32)
           + b3_ref[...])
    out_ref[...] = jnp.maximum(pre, 0.0)


def _proj(h, w1a, w1b, b1r):
    nb = N_NODES // _NODE_BLK
    full = lambda i: (0, 0)
    return pl.pallas_call(
        _proj_body,
        grid=(nb,),
        in_specs=[pl.BlockSpec((_NODE_BLK, D), lambda i: (i, 0)),
                  pl.BlockSpec((D, D), full),
                  pl.BlockSpec((D, D), full),
                  pl.BlockSpec((1, D), full)],
        out_specs=[pl.BlockSpec((_NODE_BLK, D), lambda i: (i, 0)),
                   pl.BlockSpec((_NODE_BLK, D), lambda i: (i, 0))],
        out_shape=[jax.ShapeDtypeStruct((N_NODES, D), jnp.float32),
                   jax.ShapeDtypeStruct((N_NODES, D), jnp.float32)],
    )(h, w1a, w1b, b1r)


def _msg(ssrc, sdst, e, w1c, w2, b2r):
    nb = N_EDGES // _EDGE_BLK
    full = lambda i: (0, 0)
    return pl.pallas_call(
        _msg_body,
        grid=(nb,),
        in_specs=[pl.BlockSpec((_EDGE_BLK, D), lambda i: (i, 0)),
                  pl.BlockSpec((_EDGE_BLK, D), lambda i: (i, 0)),
                  pl.BlockSpec((_EDGE_BLK, ED), lambda i: (i, 0)),
                  pl.BlockSpec((D, ED), full),
                  pl.BlockSpec((D, D), full),
                  pl.BlockSpec((1, D), full)],
        out_specs=pl.BlockSpec((_EDGE_BLK, D), lambda i: (i, 0)),
        out_shape=jax.ShapeDtypeStruct((N_EDGES, D), jnp.float32),
    )(ssrc, sdst, e, w1c, w2, b2r)


def _update(h, parts, w3a, w3b, b3r):
    nb = N_NODES // _NODE_BLK
    full = lambda i: (0, 0)
    nparts = parts.shape[0]
    return pl.pallas_call(
        _update_body,
        grid=(nb,),
        in_specs=[pl.BlockSpec((_NODE_BLK, D), lambda i: (i, 0)),
                  pl.BlockSpec((nparts, _NODE_BLK, D), lambda i: (0, i, 0)),
                  pl.BlockSpec((D, D), full),
                  pl.BlockSpec((D, D), full),
                  pl.BlockSpec((1, D), full)],
        out_specs=pl.BlockSpec((_NODE_BLK, D), lambda i: (i, 0)),
        out_shape=jax.ShapeDtypeStruct((N_NODES, D), jnp.float32),
    )(h, parts, w3a, w3b, b3r)


def _sc_info():
    try:
        info = plsc.get_sparse_core_info()
        return info.num_cores, info.num_subcores
    except Exception:
        return 2, 16


@functools.cache
def _make_gather():
    nc, ns = _sc_info()
    nw = nc * ns
    epw = N_EDGES // nw
    nch = epw // _CH
    mesh = plsc.VectorSubcoreMesh(core_axis_name="c", subcore_axis_name="s")

    @functools.partial(
        pl.kernel, mesh=mesh,
        out_type=(jax.ShapeDtypeStruct((N_EDGES, D), jnp.float32),
                  jax.ShapeDtypeStruct((N_EDGES, D), jnp.float32)),
        scratch_types=[pltpu.VMEM((_CH,), jnp.int32),
                       pltpu.VMEM((_CH,), jnp.int32),
                       pltpu.VMEM((_CH, D), jnp.float32),
                       pltpu.VMEM((_CH, D), jnp.float32),
                       pltpu.SemaphoreType.DMA,
                       pltpu.SemaphoreType.DMA])
    def gather_k(a_hbm, b_hbm, src_hbm, dst_hbm, osrc_hbm, odst_hbm,
                 sidx, didx, abuf, bbuf, sema, semb):
        wid = lax.axis_index("s") * nc + lax.axis_index("c")
        base = wid * epw

        def body(i, carry):
            off = base + i * _CH
            pltpu.sync_copy(src_hbm.at[pl.ds(off, _CH)], sidx)
            pltpu.sync_copy(dst_hbm.at[pl.ds(off, _CH)], didx)
            cpa = pltpu.async_copy(a_hbm.at[sidx], abuf, sema)
            cpb = pltpu.async_copy(b_hbm.at[didx], bbuf, semb)
            cpa.wait()
            cpb.wait()
            pltpu.sync_copy(abuf, osrc_hbm.at[pl.ds(off, _CH)])
            pltpu.sync_copy(bbuf, odst_hbm.at[pl.ds(off, _CH)])
            return carry

        lax.fori_loop(0, nch, body, 0)

    return gather_k


@functools.cache
def _make_scatter():
    nc, ns = _sc_info()
    nw = nc * ns
    epw = N_EDGES // nw
    nch = epw // _CH
    rpt = -(-N_NODES // (ns * _CH)) * _CH
    npad = rpt * ns
    spt = rpt // _CH
    mesh = plsc.VectorSubcoreMesh(core_axis_name="c", subcore_axis_name="s")

    @functools.partial(
        pl.kernel, mesh=mesh,
        out_type=jax.ShapeDtypeStruct((nc, npad, D), jnp.float32),
        scratch_types=[pltpu.VMEM((_CH,), jnp.int32),
                       pltpu.VMEM((_CH, D), jnp.float32),
                       pltpu.VMEM_SHARED((npad, D), jnp.float32)])
    def scatter_k(msg_hbm, dst_hbm, zeros_hbm, out_hbm, didx, mbuf, acc):
        c = lax.axis_index("c")
        s = lax.axis_index("s")

        pltpu.sync_copy(zeros_hbm, mbuf)
        for k in range(spt):
            pltpu.sync_copy(mbuf, acc.at[pl.ds(s * rpt + k * _CH, _CH)])
        plsc.subcore_barrier()

        wid = s * nc + c
        base = wid * epw

        def body(i, carry):
            off = base + i * _CH
            pltpu.sync_copy(dst_hbm.at[pl.ds(off, _CH)], didx)
            pltpu.sync_copy(msg_hbm.at[pl.ds(off, _CH)], mbuf)
            pltpu.sync_copy(mbuf, acc.at[didx], add=True)
            return carry

        lax.fori_loop(0, nch, body, 0)
        plsc.subcore_barrier()

        for k in range(spt):
            pltpu.sync_copy(acc.at[pl.ds(s * rpt + k * _CH, _CH)], mbuf)
            pltpu.sync_copy(mbuf, out_hbm.at[c, pl.ds(s * rpt + k * _CH, _CH)])

    return scatter_k


def kernel(h, edge_index, e, W1, b1, W2, b2, W3, b3):
    src = edge_index[0]
    dst = edge_index[1]
    w1a = W1[:, :D]
    w1b = W1[:, D:2 * D]
    w1c = W1[:, 2 * D:]
    w3a = W3[:, :D]
    w3b = W3[:, D:]
    b1r = b1.reshape(1, D)
    b2r = b2.reshape(1, D)
    b3r = b3.reshape(1, D)

    a, b = _proj(h, w1a, w1b, b1r)
    ssrc, sdst = _make_gather()(a, b, src, dst)
    msgs = _msg(ssrc, sdst, e, w1c, W2, b2r)
    zrows = jnp.zeros((_CH, D), jnp.float32)
    parts = _make_scatter()(msgs, dst, zrows)
    return _update(h, parts, w3a, w3b, b3r)


# speedup vs baseline: 3.1681x; 1.1760x over previous
import functools

import jax
import jax.numpy as jnp
from jax import lax
from jax.experimental import pallas as pl
from jax.experimental.pallas import tpu as pltpu
from jax.experimental.pallas import tpu_sc as plsc

N_NODES = 10000
N_EDGES = 320000
D = 128
ED = 16

_NODE_BLK = 2000
_EDGE_BLK = 2000
_CH = 80


def _proj_body(h_ref, w1a_ref, w1b_ref, b1_ref, a_ref, b_ref):
    h = h_ref[...]
    a_ref[...] = lax.dot_general(h, w1a_ref[...], (((1,), (1,)), ((), ())),
                                 preferred_element_type=jnp.float32) + b1_ref[...]
    b_ref[...] = lax.dot_general(h, w1b_ref[...], (((1,), (1,)), ((), ())),
                                 preferred_element_type=jnp.float32)


def _msg_body(ssrc_ref, sdst_ref, e_ref, w1c_ref, w2_ref, b2_ref, out_ref):
    pre = ssrc_ref[...] + sdst_ref[...] + lax.dot_general(
        e_ref[...], w1c_ref[...], (((1,), (1,)), ((), ())),
        preferred_element_type=jnp.float32)
    hidden = jnp.maximum(pre, 0.0)
    out_ref[...] = lax.dot_general(hidden, w2_ref[...], (((1,), (1,)), ((), ())),
                                   preferred_element_type=jnp.float32) + b2_ref[...]


def _update_body(h_ref, p_ref, w3a_ref, w3b_ref, b3_ref, out_ref):
    agg = p_ref[0] + p_ref[1]
    pre = (lax.dot_general(h_ref[...], w3a_ref[...], (((1,), (1,)), ((), ())),
                           preferred_element_type=jnp.float32)
           + lax.dot_general(agg, w3b_ref[...], (((1,), (1,)), ((), ())),
                             preferred_element_type=jnp.float32)
           + b3_ref[...])
    out_ref[...] = jnp.maximum(pre, 0.0)


def _proj(h, w1a, w1b, b1r):
    nb = N_NODES // _NODE_BLK
    full = lambda i: (0, 0)
    return pl.pallas_call(
        _proj_body,
        grid=(nb,),
        in_specs=[pl.BlockSpec((_NODE_BLK, D), lambda i: (i, 0)),
                  pl.BlockSpec((D, D), full),
                  pl.BlockSpec((D, D), full),
                  pl.BlockSpec((1, D), full)],
        out_specs=[pl.BlockSpec((_NODE_BLK, D), lambda i: (i, 0)),
                   pl.BlockSpec((_NODE_BLK, D), lambda i: (i, 0))],
        out_shape=[jax.ShapeDtypeStruct((N_NODES, D), jnp.float32),
                   jax.ShapeDtypeStruct((N_NODES, D), jnp.float32)],
    )(h, w1a, w1b, b1r)


def _msg(ssrc, sdst, e, w1c, w2, b2r):
    nb = N_EDGES // _EDGE_BLK
    full = lambda i: (0, 0)
    return pl.pallas_call(
        _msg_body,
        grid=(nb,),
        in_specs=[pl.BlockSpec((_EDGE_BLK, D), lambda i: (i, 0)),
                  pl.BlockSpec((_EDGE_BLK, D), lambda i: (i, 0)),
                  pl.BlockSpec((_EDGE_BLK, ED), lambda i: (i, 0)),
                  pl.BlockSpec((D, ED), full),
                  pl.BlockSpec((D, D), full),
                  pl.BlockSpec((1, D), full)],
        out_specs=pl.BlockSpec((_EDGE_BLK, D), lambda i: (i, 0)),
        out_shape=jax.ShapeDtypeStruct((N_EDGES, D), jnp.float32),
    )(ssrc, sdst, e, w1c, w2, b2r)


def _update(h, parts, w3a, w3b, b3r):
    nb = N_NODES // _NODE_BLK
    full = lambda i: (0, 0)
    nparts = parts.shape[0]
    return pl.pallas_call(
        _update_body,
        grid=(nb,),
        in_specs=[pl.BlockSpec((_NODE_BLK, D), lambda i: (i, 0)),
                  pl.BlockSpec((nparts, _NODE_BLK, D), lambda i: (0, i, 0)),
                  pl.BlockSpec((D, D), full),
                  pl.BlockSpec((D, D), full),
                  pl.BlockSpec((1, D), full)],
        out_specs=pl.BlockSpec((_NODE_BLK, D), lambda i: (i, 0)),
        out_shape=jax.ShapeDtypeStruct((N_NODES, D), jnp.float32),
    )(h, parts, w3a, w3b, b3r)


def _sc_info():
    try:
        info = plsc.get_sparse_core_info()
        return info.num_cores, info.num_subcores
    except Exception:
        return 2, 16


_GCH = 40
_GNB = 5


@functools.cache
def _make_gather():
    nc, ns = _sc_info()
    nw = nc * ns
    epw = N_EDGES // nw
    nch = epw // _GCH
    nrounds = nch // _GNB
    mesh = plsc.VectorSubcoreMesh(core_axis_name="c", subcore_axis_name="s")

    @functools.partial(
        pl.kernel, mesh=mesh,
        out_type=(jax.ShapeDtypeStruct((N_EDGES, D), jnp.float32),
                  jax.ShapeDtypeStruct((N_EDGES, D), jnp.float32)),
        scratch_types=[pltpu.VMEM((epw,), jnp.int32),
                       pltpu.VMEM((epw,), jnp.int32),
                       pltpu.VMEM((2, _GNB, _GCH, D), jnp.float32),
                       pltpu.VMEM((2, _GNB, _GCH, D), jnp.float32),
                       pltpu.SemaphoreType.DMA((2, _GNB)),
                       pltpu.SemaphoreType.DMA((2, _GNB))])
    def gather_k(a_hbm, b_hbm, src_hbm, dst_hbm, osrc_hbm, odst_hbm,
                 sidx, didx, abuf, bbuf, gsem, wsem):
        wid = lax.axis_index("s") * nc + lax.axis_index("c")
        base = wid * epw
        pltpu.sync_copy(src_hbm.at[pl.ds(base, epw)], sidx)
        pltpu.sync_copy(dst_hbm.at[pl.ds(base, epw)], didx)

        def fire_gather(p, b, g):
            ioff = g * _GCH
            pltpu.async_copy(a_hbm.at[sidx.at[pl.ds(ioff, _GCH)]],
                             abuf.at[p, b], gsem.at[p, b])
            pltpu.async_copy(b_hbm.at[didx.at[pl.ds(ioff, _GCH)]],
                             bbuf.at[p, b], gsem.at[p, b])

        def wait_gather(p, b):
            pltpu.make_async_copy(a_hbm.at[pl.ds(0, _GCH)], abuf.at[p, b],
                                  gsem.at[p, b]).wait()
            pltpu.make_async_copy(b_hbm.at[pl.ds(0, _GCH)], bbuf.at[p, b],
                                  gsem.at[p, b]).wait()

        def fire_write(p, b, g):
            off = base + g * _GCH
            pltpu.async_copy(abuf.at[p, b], osrc_hbm.at[pl.ds(off, _GCH)],
                             wsem.at[p, b])
            pltpu.async_copy(bbuf.at[p, b], odst_hbm.at[pl.ds(off, _GCH)],
                             wsem.at[p, b])

        def wait_write(p, b):
            pltpu.make_async_copy(abuf.at[p, b],
                                  osrc_hbm.at[pl.ds(base, _GCH)],
                                  wsem.at[p, b]).wait()
            pltpu.make_async_copy(bbuf.at[p, b],
                                  odst_hbm.at[pl.ds(base, _GCH)],
                                  wsem.at[p, b]).wait()

        for b in range(_GNB):
            fire_gather(0, b, b)

        def super_round(r2, carry):
            for p in range(2):
                r = 2 * r2 + p
                for b in range(_GNB):
                    g = r * _GNB + b
                    wait_gather(p, b)
                    fire_write(p, b, g)
                    @pl.when(r >= 1)
                    def _():
                        wait_write(1 - p, b)

                    @pl.when(r < nrounds - 1)
                    def _():
                        fire_gather(1 - p, b, g + _GNB)
            return carry

        lax.fori_loop(0, nrounds // 2, super_round, 0)
        for b in range(_GNB):
            wait_write((nrounds - 1) % 2, b)

    return gather_k


@functools.cache
def _make_scatter():
    nc, ns = _sc_info()
    nw = nc * ns
    epw = N_EDGES // nw
    nch = epw // _CH
    rpt = -(-N_NODES // (ns * _CH)) * _CH
    npad = rpt * ns
    spt = rpt // _CH
    mesh = plsc.VectorSubcoreMesh(core_axis_name="c", subcore_axis_name="s")

    @functools.partial(
        pl.kernel, mesh=mesh,
        out_type=jax.ShapeDtypeStruct((nc, npad, D), jnp.float32),
        scratch_types=[pltpu.VMEM((_CH,), jnp.int32),
                       pltpu.VMEM((_CH, D), jnp.float32),
                       pltpu.VMEM_SHARED((npad, D), jnp.float32)])
    def scatter_k(msg_hbm, dst_hbm, zeros_hbm, out_hbm, didx, mbuf, acc):
        c = lax.axis_index("c")
        s = lax.axis_index("s")

        pltpu.sync_copy(zeros_hbm, mbuf)
        for k in range(spt):
            pltpu.sync_copy(mbuf, acc.at[pl.ds(s * rpt + k * _CH, _CH)])
        plsc.subcore_barrier()

        wid = s * nc + c
        base = wid * epw

        def body(i, carry):
            off = base + i * _CH
            pltpu.sync_copy(dst_hbm.at[pl.ds(off, _CH)], didx)
            pltpu.sync_copy(msg_hbm.at[pl.ds(off, _CH)], mbuf)
            pltpu.sync_copy(mbuf, acc.at[didx], add=True)
            return carry

        lax.fori_loop(0, nch, body, 0)
        plsc.subcore_barrier()

        for k in range(spt):
            pltpu.sync_copy(acc.at[pl.ds(s * rpt + k * _CH, _CH)], mbuf)
            pltpu.sync_copy(mbuf, out_hbm.at[c, pl.ds(s * rpt + k * _CH, _CH)])

    return scatter_k


def kernel(h, edge_index, e, W1, b1, W2, b2, W3, b3):
    src = edge_index[0]
    dst = edge_index[1]
    w1a = W1[:, :D]
    w1b = W1[:, D:2 * D]
    w1c = W1[:, 2 * D:]
    w3a = W3[:, :D]
    w3b = W3[:, D:]
    b1r = b1.reshape(1, D)
    b2r = b2.reshape(1, D)
    b3r = b3.reshape(1, D)

    a, b = _proj(h, w1a, w1b, b1r)
    ssrc, sdst = _make_gather()(a, b, src, dst)
    msgs = _msg(ssrc, sdst, e, w1c, W2, b2r)
    zrows = jnp.zeros((_CH, D), jnp.float32)
    parts = _make_scatter()(msgs, dst, zrows)
    return _update(h, parts, w3a, w3b, b3r)


# speedup vs baseline: 3.7935x; 1.1974x over previous
import functools

import jax
import jax.numpy as jnp
from jax import lax
from jax.experimental import pallas as pl
from jax.experimental.pallas import tpu as pltpu
from jax.experimental.pallas import tpu_sc as plsc

N_NODES = 10000
N_EDGES = 320000
D = 128
ED = 16

_NODE_BLK = 2000
_EDGE_BLK = 2000
_CH = 80


def _proj_body(h_ref, w1a_ref, w1b_ref, b1_ref, a_ref, b_ref):
    h = h_ref[...]
    a_ref[...] = lax.dot_general(h, w1a_ref[...], (((1,), (1,)), ((), ())),
                                 preferred_element_type=jnp.float32) + b1_ref[...]
    b_ref[...] = lax.dot_general(h, w1b_ref[...], (((1,), (1,)), ((), ())),
                                 preferred_element_type=jnp.float32)


def _msg_body(ssrc_ref, sdst_ref, e_ref, w1c_ref, w2_ref, b2_ref, out_ref):
    pre = ssrc_ref[...] + sdst_ref[...] + lax.dot_general(
        e_ref[...], w1c_ref[...], (((1,), (1,)), ((), ())),
        preferred_element_type=jnp.float32)
    hidden = jnp.maximum(pre, 0.0)
    out_ref[...] = lax.dot_general(hidden, w2_ref[...], (((1,), (1,)), ((), ())),
                                   preferred_element_type=jnp.float32) + b2_ref[...]


def _update_body(h_ref, p_ref, w3a_ref, w3b_ref, b3_ref, out_ref):
    agg = p_ref[0] + p_ref[1]
    pre = (lax.dot_general(h_ref[...], w3a_ref[...], (((1,), (1,)), ((), ())),
                           preferred_element_type=jnp.float32)
           + lax.dot_general(agg, w3b_ref[...], (((1,), (1,)), ((), ())),
                             preferred_element_type=jnp.float32)
           + b3_ref[...])
    out_ref[...] = jnp.maximum(pre, 0.0)


def _proj(h, w1a, w1b, b1r):
    nb = N_NODES // _NODE_BLK
    full = lambda i: (0, 0)
    return pl.pallas_call(
        _proj_body,
        grid=(nb,),
        in_specs=[pl.BlockSpec((_NODE_BLK, D), lambda i: (i, 0)),
                  pl.BlockSpec((D, D), full),
                  pl.BlockSpec((D, D), full),
                  pl.BlockSpec((1, D), full)],
        out_specs=[pl.BlockSpec((_NODE_BLK, D), lambda i: (i, 0)),
                   pl.BlockSpec((_NODE_BLK, D), lambda i: (i, 0))],
        out_shape=[jax.ShapeDtypeStruct((N_NODES, D), jnp.float32),
                   jax.ShapeDtypeStruct((N_NODES, D), jnp.float32)],
    )(h, w1a, w1b, b1r)


def _msg(ssrc, sdst, e, w1c, w2, b2r):
    nb = N_EDGES // _EDGE_BLK
    full = lambda i: (0, 0)
    return pl.pallas_call(
        _msg_body,
        grid=(nb,),
        in_specs=[pl.BlockSpec((_EDGE_BLK, D), lambda i: (i, 0)),
                  pl.BlockSpec((_EDGE_BLK, D), lambda i: (i, 0)),
                  pl.BlockSpec((_EDGE_BLK, ED), lambda i: (i, 0)),
                  pl.BlockSpec((D, ED), full),
                  pl.BlockSpec((D, D), full),
                  pl.BlockSpec((1, D), full)],
        out_specs=pl.BlockSpec((_EDGE_BLK, D), lambda i: (i, 0)),
        out_shape=jax.ShapeDtypeStruct((N_EDGES, D), jnp.float32),
    )(ssrc, sdst, e, w1c, w2, b2r)


def _update(h, parts, w3a, w3b, b3r):
    nb = N_NODES // _NODE_BLK
    full = lambda i: (0, 0)
    nparts = parts.shape[0]
    return pl.pallas_call(
        _update_body,
        grid=(nb,),
        in_specs=[pl.BlockSpec((_NODE_BLK, D), lambda i: (i, 0)),
                  pl.BlockSpec((nparts, _NODE_BLK, D), lambda i: (0, i, 0)),
                  pl.BlockSpec((D, D), full),
                  pl.BlockSpec((D, D), full),
                  pl.BlockSpec((1, D), full)],
        out_specs=pl.BlockSpec((_NODE_BLK, D), lambda i: (i, 0)),
        out_shape=jax.ShapeDtypeStruct((N_NODES, D), jnp.float32),
    )(h, parts, w3a, w3b, b3r)


def _sc_info():
    try:
        info = plsc.get_sparse_core_info()
        return info.num_cores, info.num_subcores
    except Exception:
        return 2, 16


_GCH = 40
_GNB = 5


@functools.cache
def _make_gather():
    nc, ns = _sc_info()
    nw = nc * ns
    epw = N_EDGES // nw
    nch = epw // _GCH
    nrounds = nch // _GNB
    mesh = plsc.VectorSubcoreMesh(core_axis_name="c", subcore_axis_name="s")

    @functools.partial(
        pl.kernel, mesh=mesh,
        out_type=(jax.ShapeDtypeStruct((N_EDGES, D), jnp.float32),
                  jax.ShapeDtypeStruct((N_EDGES, D), jnp.float32)),
        scratch_types=[pltpu.VMEM((epw,), jnp.int32),
                       pltpu.VMEM((epw,), jnp.int32),
                       pltpu.VMEM((2, _GNB, _GCH, D), jnp.float32),
                       pltpu.VMEM((2, _GNB, _GCH, D), jnp.float32),
                       pltpu.SemaphoreType.DMA((2, _GNB)),
                       pltpu.SemaphoreType.DMA((2, _GNB))])
    def gather_k(a_hbm, b_hbm, src_hbm, dst_hbm, osrc_hbm, odst_hbm,
                 sidx, didx, abuf, bbuf, gsem, wsem):
        wid = lax.axis_index("s") * nc + lax.axis_index("c")
        base = wid * epw
        pltpu.sync_copy(src_hbm.at[pl.ds(base, epw)], sidx)
        pltpu.sync_copy(dst_hbm.at[pl.ds(base, epw)], didx)

        def fire_gather(p, b, g):
            ioff = g * _GCH
            pltpu.async_copy(a_hbm.at[sidx.at[pl.ds(ioff, _GCH)]],
                             abuf.at[p, b], gsem.at[p, b])
            pltpu.async_copy(b_hbm.at[didx.at[pl.ds(ioff, _GCH)]],
                             bbuf.at[p, b], gsem.at[p, b])

        def wait_gather(p, b):
            pltpu.make_async_copy(a_hbm.at[pl.ds(0, _GCH)], abuf.at[p, b],
                                  gsem.at[p, b]).wait()
            pltpu.make_async_copy(b_hbm.at[pl.ds(0, _GCH)], bbuf.at[p, b],
                                  gsem.at[p, b]).wait()

        def fire_write(p, b, g):
            off = base + g * _GCH
            pltpu.async_copy(abuf.at[p, b], osrc_hbm.at[pl.ds(off, _GCH)],
                             wsem.at[p, b])
            pltpu.async_copy(bbuf.at[p, b], odst_hbm.at[pl.ds(off, _GCH)],
                             wsem.at[p, b])

        def wait_write(p, b):
            pltpu.make_async_copy(abuf.at[p, b],
                                  osrc_hbm.at[pl.ds(base, _GCH)],
                                  wsem.at[p, b]).wait()
            pltpu.make_async_copy(bbuf.at[p, b],
                                  odst_hbm.at[pl.ds(base, _GCH)],
                                  wsem.at[p, b]).wait()

        for b in range(_GNB):
            fire_gather(0, b, b)

        def super_round(r2, carry):
            for p in range(2):
                r = 2 * r2 + p
                for b in range(_GNB):
                    g = r * _GNB + b
                    wait_gather(p, b)
                    fire_write(p, b, g)
                    @pl.when(r >= 1)
                    def _():
                        wait_write(1 - p, b)

                    @pl.when(r < nrounds - 1)
                    def _():
                        fire_gather(1 - p, b, g + _GNB)
            return carry

        lax.fori_loop(0, nrounds // 2, super_round, 0)
        for b in range(_GNB):
            wait_write((nrounds - 1) % 2, b)

    return gather_k


@functools.cache
def _make_scatter():
    nc, ns = _sc_info()
    nw = nc * ns
    epw = N_EDGES // nw
    nch = epw // _CH
    rpt = -(-N_NODES // (ns * _CH)) * _CH
    npad = rpt * ns
    spt = rpt // _CH
    mesh = plsc.VectorSubcoreMesh(core_axis_name="c", subcore_axis_name="s")

    snb = 2
    nrounds = nch // snb

    @functools.partial(
        pl.kernel, mesh=mesh,
        out_type=jax.ShapeDtypeStruct((nc, npad, D), jnp.float32),
        scratch_types=[pltpu.VMEM((2, snb, _CH), jnp.int32),
                       pltpu.VMEM((2, snb, _CH, D), jnp.float32),
                       pltpu.SemaphoreType.DMA((2, snb)),
                       pltpu.SemaphoreType.DMA((2, snb)),
                       pltpu.VMEM_SHARED((npad, D), jnp.float32)])
    def scatter_k(msg_hbm, dst_hbm, zeros_hbm, out_hbm, didx, mbuf,
                  lsem, ssem, acc):
        c = lax.axis_index("c")
        s = lax.axis_index("s")
        wid = s * nc + c
        base = wid * epw

        pltpu.sync_copy(zeros_hbm, mbuf.at[0, 0])
        for k in range(spt):
            pltpu.sync_copy(mbuf.at[0, 0], acc.at[pl.ds(s * rpt + k * _CH, _CH)])
        plsc.subcore_barrier()

        def fire_load(p, b, g):
            off = base + g * _CH
            pltpu.async_copy(dst_hbm.at[pl.ds(off, _CH)], didx.at[p, b],
                             lsem.at[p, b])
            pltpu.async_copy(msg_hbm.at[pl.ds(off, _CH)], mbuf.at[p, b],
                             lsem.at[p, b])

        def wait_load(p, b):
            pltpu.make_async_copy(dst_hbm.at[pl.ds(base, _CH)],
                                  didx.at[p, b], lsem.at[p, b]).wait()
            pltpu.make_async_copy(msg_hbm.at[pl.ds(base, _CH)],
                                  mbuf.at[p, b], lsem.at[p, b]).wait()

        def fire_scatter(p, b):
            pltpu.async_copy(mbuf.at[p, b], acc.at[didx.at[p, b]],
                             ssem.at[p, b], add=True)

        def wait_scatter(p, b):
            pltpu.make_async_copy(mbuf.at[p, b], acc.at[didx.at[p, b]],
                                  ssem.at[p, b]).wait()

        for b in range(snb):
            fire_load(0, b, b)

        def super_round(r2, carry):
            for p in range(2):
                r = 2 * r2 + p
                for b in range(snb):
                    g = r * snb + b
                    wait_load(p, b)
                    fire_scatter(p, b)

                    @pl.when(r >= 1)
                    def _():
                        wait_scatter(1 - p, b)

                    @pl.when(r < nrounds - 1)
                    def _():
                        fire_load(1 - p, b, g + snb)
            return carry

        lax.fori_loop(0, nrounds // 2, super_round, 0)
        for b in range(snb):
            wait_scatter((nrounds - 1) % 2, b)
        for g in range(nrounds * snb, nch):
            off = base + g * _CH
            pltpu.sync_copy(dst_hbm.at[pl.ds(off, _CH)], didx.at[0, 0])
            pltpu.sync_copy(msg_hbm.at[pl.ds(off, _CH)], mbuf.at[0, 0])
            pltpu.sync_copy(mbuf.at[0, 0], acc.at[didx.at[0, 0]], add=True)

        plsc.subcore_barrier()
        for k in range(spt):
            pltpu.sync_copy(acc.at[pl.ds(s * rpt + k * _CH, _CH)], mbuf.at[0, 0])
            pltpu.sync_copy(mbuf.at[0, 0],
                            out_hbm.at[c, pl.ds(s * rpt + k * _CH, _CH)])

    return scatter_k


def kernel(h, edge_index, e, W1, b1, W2, b2, W3, b3):
    src = edge_index[0]
    dst = edge_index[1]
    w1a = W1[:, :D]
    w1b = W1[:, D:2 * D]
    w1c = W1[:, 2 * D:]
    w3a = W3[:, :D]
    w3b = W3[:, D:]
    b1r = b1.reshape(1, D)
    b2r = b2.reshape(1, D)
    b3r = b3.reshape(1, D)

    a, b = _proj(h, w1a, w1b, b1r)
    ssrc, sdst = _make_gather()(a, b, src, dst)
    msgs = _msg(ssrc, sdst, e, w1c, W2, b2r)
    zrows = jnp.zeros((_CH, D), jnp.float32)
    parts = _make_scatter()(msgs, dst, zrows)
    return _update(h, parts, w3a, w3b, b3r)
